# Initial kernel scaffold; baseline (speedup 1.0000x reference)
import functools

import jax
import jax.numpy as jnp
from jax import lax
from jax.experimental import pallas as pl
from jax.experimental.pallas import tpu as pltpu
from jax.experimental.pallas import tpu_sc as plsc

_NC = 2
_NS = 16
_NW = _NC * _NS
_IDXW = 128


def _sc_gather(center_words, context_words, neg_flat, center_table,
               context_table):
    B = center_words.shape[0]
    BK = neg_flat.shape[0]
    D = center_table.shape[1]
    rows_w = B // _NW
    nrows_w = BK // _NW
    jc = rows_w // _IDXW
    jn_total = nrows_w // _IDXW
    JN = 10
    NBUF = JN * _IDXW

    cw2 = center_words.reshape(B // _IDXW, _IDXW)
    xw2 = context_words.reshape(B // _IDXW, _IDXW)
    nw2 = neg_flat.reshape(BK // _IDXW, _IDXW)

    mesh = plsc.VectorSubcoreMesh(core_axis_name="c", subcore_axis_name="s")

    @functools.partial(
        pl.kernel,
        out_type=[
            jax.ShapeDtypeStruct((B, D), jnp.float32),
            jax.ShapeDtypeStruct((B, D), jnp.float32),
            jax.ShapeDtypeStruct((BK, D), jnp.float32),
        ],
        mesh=mesh,
        scratch_types=[
            pltpu.VMEM((10, _IDXW), jnp.int32),
            pltpu.VMEM((10 * _IDXW, 64), jnp.float32),
            pltpu.SemaphoreType.DMA,
        ],
    )
    def k(cw_hbm, xw_hbm, nw_hbm, ctab_hbm, xtab_hbm, c_out, x_out, n_out,
          idx_v, rows_v, sem):
        wid = lax.axis_index("s") * _NC + lax.axis_index("c")

        def gather_block(words_hbm, idx_row0, nidx, table_hbm, out_hbm,
                         out_row0):
            pltpu.sync_copy(words_hbm.at[pl.ds(idx_row0, nidx)],
                            idx_v.at[pl.ds(0, nidx)])
            copies = []
            for j in range(nidx):
                copies.append(pltpu.async_copy(
                    table_hbm.at[idx_v.at[j]],
                    rows_v.at[pl.ds(j * _IDXW, _IDXW)], sem))
            for c in copies:
                c.wait()
            pltpu.sync_copy(rows_v.at[pl.ds(0, nidx * _IDXW)],
                            out_hbm.at[pl.ds(out_row0, nidx * _IDXW)])

        gather_block(cw_hbm, wid * jc, jc, ctab_hbm, c_out, wid * rows_w)
        gather_block(xw_hbm, wid * jc, jc, xtab_hbm, x_out, wid * rows_w)
        for s in range(jn_total // JN):
            gather_block(nw_hbm, wid * jn_total + s * JN, JN, xtab_hbm,
                         n_out, (wid * jn_total + s * JN) * _IDXW)

    return k(cw2, xw2, nw2, center_table, context_table)


def _log_sigmoid(v):
    return jnp.minimum(v, 0.0) - jnp.log1p(jnp.exp(-jnp.abs(v)))


def _tc_loss(center_e, context_e, neg_e_flat, B, K, D, blk):
    nsteps = B // blk

    def body(c_ref, x_ref, n_ref, o_ref):
        i = pl.program_id(0)
        c = c_ref[...]
        x = x_ref[...]
        pos = jnp.sum(c * x, axis=1)
        n = n_ref[...].reshape(blk, K, D)
        neg = jnp.sum(n * c[:, None, :], axis=2)
        total = jnp.sum(_log_sigmoid(pos)) + jnp.sum(_log_sigmoid(-neg))
        prev = jnp.where(i == 0, 0.0, o_ref[0, 0])
        acc = prev + total
        o_ref[0, 0] = jnp.where(i == nsteps - 1, -acc / B, acc)

    out = pl.pallas_call(
        body,
        grid=(nsteps,),
        in_specs=[
            pl.BlockSpec((blk, D), lambda i: (i, 0)),
            pl.BlockSpec((blk, D), lambda i: (i, 0)),
            pl.BlockSpec((blk * K, D), lambda i: (i, 0)),
        ],
        out_specs=pl.BlockSpec((1, 1), lambda i: (0, 0)),
        out_shape=jax.ShapeDtypeStruct((1, 1), jnp.float32),
    )(center_e, context_e, neg_e_flat)
    return out[0, 0]


def kernel(center_words, context_words, negative_words, center_table,
           context_table):
    B = center_words.shape[0]
    K = negative_words.shape[1]
    D = center_table.shape[1]
    neg_flat = negative_words.reshape(B * K)
    c_e, x_e, n_e = _sc_gather(center_words, context_words, neg_flat,
                               center_table, context_table)
    return _tc_loss(c_e, x_e, n_e, B, K, D, blk=1024)


# speedup vs baseline: 4.2705x; 4.2705x over previous
import functools

import jax
import jax.numpy as jnp
from jax import lax
from jax.experimental import pallas as pl
from jax.experimental.pallas import tpu as pltpu
from jax.experimental.pallas import tpu_sc as plsc

_NC = 2
_NS = 16
_NW = _NC * _NS
_IDXW = 128


def _sc_gather(center_words, context_words, neg_flat, center_table,
               context_table):
    B = center_words.shape[0]
    BK = neg_flat.shape[0]
    D = center_table.shape[1]
    rows_w = B // _NW
    nrows_w = BK // _NW
    jc = rows_w // _IDXW
    jn_total = nrows_w // _IDXW
    JN = 8
    NBUF = JN * _IDXW
    nidx_w = 2 * rows_w + nrows_w

    mesh = plsc.VectorSubcoreMesh(core_axis_name="c", subcore_axis_name="s")

    @functools.partial(
        pl.kernel,
        out_type=[
            jax.ShapeDtypeStruct((B, D), jnp.float32),
            jax.ShapeDtypeStruct((B, D), jnp.float32),
            jax.ShapeDtypeStruct((BK, D), jnp.float32),
        ],
        mesh=mesh,
        compiler_params=pltpu.CompilerParams(use_tc_tiling_on_sc=False),
        scratch_types=[
            pltpu.VMEM((nidx_w,), jnp.int32),
            pltpu.VMEM((NBUF, D), jnp.float32),
            pltpu.SemaphoreType.DMA,
        ],
    )
    def k(cw_hbm, xw_hbm, nw_hbm, ctab_hbm, xtab_hbm, c_out, x_out, n_out,
          idx_v, rows_v, sem):
        wid = lax.axis_index("s") * _NC + lax.axis_index("c")

        pltpu.sync_copy(cw_hbm.at[pl.ds(wid * rows_w, rows_w)],
                        idx_v.at[pl.ds(0, rows_w)])
        pltpu.sync_copy(xw_hbm.at[pl.ds(wid * rows_w, rows_w)],
                        idx_v.at[pl.ds(rows_w, rows_w)])
        pltpu.sync_copy(nw_hbm.at[pl.ds(wid * nrows_w, nrows_w)],
                        idx_v.at[pl.ds(2 * rows_w, nrows_w)])

        def gather_block(idx0, ngather, table_hbm, out_hbm, out_row0):
            copies = []
            for j in range(ngather):
                copies.append(pltpu.async_copy(
                    table_hbm.at[idx_v.at[pl.ds(idx0 + j * _IDXW, _IDXW)]],
                    rows_v.at[pl.ds(j * _IDXW, _IDXW)], sem))
            for c in copies:
                c.wait()
            pltpu.sync_copy(rows_v.at[pl.ds(0, ngather * _IDXW)],
                            out_hbm.at[pl.ds(out_row0, ngather * _IDXW)])

        gather_block(0, jc, ctab_hbm, c_out, wid * rows_w)
        gather_block(rows_w, jc, xtab_hbm, x_out, wid * rows_w)
        for s in range(jn_total // JN):
            gather_block(2 * rows_w + s * NBUF, JN, xtab_hbm, n_out,
                         wid * nrows_w + s * NBUF)

    return k(center_words, context_words, neg_flat, center_table,
             context_table)


def _log_sigmoid(v):
    return jnp.minimum(v, 0.0) - jnp.log1p(jnp.exp(-jnp.abs(v)))


def _tc_loss(center_e, context_e, neg_e_flat, B, K, D, blk):
    nsteps = B // blk

    def body(c_ref, x_ref, n_ref, o_ref):
        i = pl.program_id(0)
        c = c_ref[...]
        x = x_ref[...]
        pos = jnp.sum(c * x, axis=1)
        n = n_ref[...].reshape(blk, K, D)
        neg = jnp.sum(n * c[:, None, :], axis=2)
        total = jnp.sum(_log_sigmoid(pos)) + jnp.sum(_log_sigmoid(-neg))
        prev = jnp.where(i == 0, 0.0, o_ref[0, 0])
        acc = prev + total
        o_ref[0, 0] = jnp.where(i == nsteps - 1, -acc / B, acc)

    out = pl.pallas_call(
        body,
        grid=(nsteps,),
        in_specs=[
            pl.BlockSpec((blk, D), lambda i: (i, 0)),
            pl.BlockSpec((blk, D), lambda i: (i, 0)),
            pl.BlockSpec((blk * K, D), lambda i: (i, 0)),
        ],
        out_specs=pl.BlockSpec(memory_space=pltpu.SMEM),
        out_shape=jax.ShapeDtypeStruct((1, 1), jnp.float32),
    )(center_e, context_e, neg_e_flat)
    return out[0, 0]


def kernel(center_words, context_words, negative_words, center_table,
           context_table):
    B = center_words.shape[0]
    K = negative_words.shape[1]
    D = center_table.shape[1]
    neg_flat = negative_words.reshape(B * K)
    c_e, x_e, n_e = _sc_gather(center_words, context_words, neg_flat,
                               center_table, context_table)
    return _tc_loss(c_e, x_e, n_e, B, K, D, blk=1024)


# speedup vs baseline: 5.0311x; 1.1781x over previous
import functools

import jax
import jax.numpy as jnp
from jax import lax
from jax.experimental import pallas as pl
from jax.experimental.pallas import tpu as pltpu
from jax.experimental.pallas import tpu_sc as plsc

_NC = 2
_NS = 16
_NW = _NC * _NS
_IDXW = 128
_CHUNK = 32
_TSTRIDE = 25


def _sc_scores(center_words, context_words, neg_flat, center_table,
               context_table, K, D):
    B = center_words.shape[0]
    BK = neg_flat.shape[0]
    rows_w = B // _NW
    nrows_w = BK // _NW
    nchunks = rows_w // _CHUNK
    nneg_g = _CHUNK * K // _IDXW
    nidx_w = 2 * rows_w + nrows_w
    nscore_w = rows_w * (K + 1)

    mesh = plsc.VectorSubcoreMesh(core_axis_name="c", subcore_axis_name="s")

    @functools.partial(
        pl.kernel,
        out_type=jax.ShapeDtypeStruct((B * (K + 1),), jnp.float32),
        mesh=mesh,
        compiler_params=pltpu.CompilerParams(use_tc_tiling_on_sc=False,
                                             needs_layout_passes=False),
        scratch_types=[
            pltpu.VMEM((nidx_w,), jnp.int32),
            pltpu.VMEM((_CHUNK, D), jnp.float32),
            pltpu.VMEM((_CHUNK, D), jnp.float32),
            pltpu.VMEM((_CHUNK * K, D), jnp.float32),
            pltpu.VMEM((16 * _TSTRIDE + 16,), jnp.float32),
            pltpu.VMEM((nscore_w + 16,), jnp.float32),
            pltpu.SemaphoreType.DMA,
        ],
    )
    def k(cw_hbm, xw_hbm, nw_hbm, ctab_hbm, xtab_hbm, out_hbm,
          idx_v, c_v, x_v, n_v, tbuf, scores_v, sem):
        wid = lax.axis_index("s") * _NC + lax.axis_index("c")
        iota = lax.iota(jnp.int32, 16)

        pltpu.sync_copy(cw_hbm.at[pl.ds(wid * rows_w, rows_w)],
                        idx_v.at[pl.ds(0, rows_w)])
        pltpu.sync_copy(xw_hbm.at[pl.ds(wid * rows_w, rows_w)],
                        idx_v.at[pl.ds(rows_w, rows_w)])
        pltpu.sync_copy(nw_hbm.at[pl.ds(wid * nrows_w, nrows_w)],
                        idx_v.at[pl.ds(2 * rows_w, nrows_w)])

        def chunk_body(s, carry):
            copies = [
                pltpu.async_copy(
                    ctab_hbm.at[idx_v.at[pl.ds(s * _CHUNK, _CHUNK)]],
                    c_v, sem),
                pltpu.async_copy(
                    xtab_hbm.at[idx_v.at[pl.ds(rows_w + s * _CHUNK,
                                               _CHUNK)]],
                    x_v, sem),
            ]
            for q in range(nneg_g):
                copies.append(pltpu.async_copy(
                    xtab_hbm.at[idx_v.at[pl.ds(
                        2 * rows_w + s * _CHUNK * K + q * _IDXW, _IDXW)]],
                    n_v.at[pl.ds(q * _IDXW, _IDXW)], sem))
            for c in copies:
                c.wait()

            def elem_body(e, carry2):
                cvec = [c_v[e, pl.ds(16 * t, 16)] for t in range(4)]
                for j in range(K + 1):
                    if j == 0:
                        r = [x_v[e, pl.ds(16 * t, 16)] for t in range(4)]
                    else:
                        row = e * K + (j - 1)
                        r = [n_v[row, pl.ds(16 * t, 16)] for t in range(4)]
                    p = (cvec[0] * r[0] + cvec[1] * r[1]
                         + cvec[2] * r[2] + cvec[3] * r[3])
                    plsc.store_scatter(tbuf, [iota * _TSTRIDE + j], p)
                s0 = plsc.load_gather(tbuf, [iota])
                s1 = plsc.load_gather(tbuf, [iota + 16])
                for l in range(1, 16):
                    s0 = s0 + plsc.load_gather(tbuf,
                                               [iota + l * _TSTRIDE])
                    s1 = s1 + plsc.load_gather(tbuf,
                                               [iota + l * _TSTRIDE + 16])
                s0 = jnp.where(iota == 0, s0, -s0)
                s1 = -s1
                base = (s * _CHUNK + e) * (K + 1)
                plsc.store_scatter(scores_v, [base + iota], s0)
                plsc.store_scatter(scores_v, [base + 16 + iota], s1)
                return carry2

            lax.fori_loop(0, _CHUNK, elem_body, 0)
            return carry

        lax.fori_loop(0, nchunks, chunk_body, 0)
        pltpu.sync_copy(scores_v.at[pl.ds(0, nscore_w)],
                        out_hbm.at[pl.ds(wid * nscore_w, nscore_w)])

    return k(center_words, context_words, neg_flat, center_table,
             context_table)


def _log_sigmoid(v):
    return jnp.minimum(v, 0.0) - jnp.log1p(jnp.exp(-jnp.abs(v)))


def _tc_loss(scores2d, B):
    def body(s_ref, o_ref):
        o_ref[0, 0] = -jnp.sum(_log_sigmoid(s_ref[...])) / B

    out = pl.pallas_call(
        body,
        out_specs=pl.BlockSpec(memory_space=pltpu.SMEM),
        out_shape=jax.ShapeDtypeStruct((1, 1), jnp.float32),
    )(scores2d)
    return out[0, 0]


def kernel(center_words, context_words, negative_words, center_table,
           context_table):
    B = center_words.shape[0]
    K = negative_words.shape[1]
    D = center_table.shape[1]
    neg_flat = negative_words.reshape(B * K)
    scores = _sc_scores(center_words, context_words, neg_flat,
                        center_table, context_table, K, D)
    scores2d = scores.reshape(B * (K + 1) // 128, 128)
    return _tc_loss(scores2d, B)


# speedup vs baseline: 5.0517x; 1.0041x over previous
import functools

import jax
import jax.numpy as jnp
from jax import lax
from jax.experimental import pallas as pl
from jax.experimental.pallas import tpu as pltpu
from jax.experimental.pallas import tpu_sc as plsc

_NC = 2
_NS = 16
_NW = _NC * _NS
_IDXW = 128
_CHUNK = 64
_TSTRIDE = 25


def _sc_scores(center_words, context_words, neg_flat, center_table,
               context_table, K, D):
    B = center_words.shape[0]
    BK = neg_flat.shape[0]
    rows_w = B // _NW
    nrows_w = BK // _NW
    nchunks = rows_w // _CHUNK
    nneg_g = _CHUNK * K // _IDXW
    nidx_w = 2 * rows_w + nrows_w
    nscore_w = rows_w * (K + 1)

    mesh = plsc.VectorSubcoreMesh(core_axis_name="c", subcore_axis_name="s")

    @functools.partial(
        pl.kernel,
        out_type=jax.ShapeDtypeStruct((B * (K + 1),), jnp.float32),
        mesh=mesh,
        compiler_params=pltpu.CompilerParams(use_tc_tiling_on_sc=False,
                                             needs_layout_passes=False),
        scratch_types=[
            pltpu.VMEM((nidx_w,), jnp.int32),
            pltpu.VMEM((_CHUNK, D), jnp.float32),
            pltpu.VMEM((_CHUNK, D), jnp.float32),
            pltpu.VMEM((_CHUNK * K, D), jnp.float32),
            pltpu.VMEM((16 * _TSTRIDE + 16,), jnp.float32),
            pltpu.VMEM((nscore_w + 16,), jnp.float32),
            pltpu.SemaphoreType.DMA,
        ],
    )
    def k(cw_hbm, xw_hbm, nw_hbm, ctab_hbm, xtab_hbm, out_hbm,
          idx_v, c_v, x_v, n_v, tbuf, scores_v, sem):
        wid = lax.axis_index("s") * _NC + lax.axis_index("c")
        iota = lax.iota(jnp.int32, 16)

        pltpu.sync_copy(cw_hbm.at[pl.ds(wid * rows_w, rows_w)],
                        idx_v.at[pl.ds(0, rows_w)])
        pltpu.sync_copy(xw_hbm.at[pl.ds(wid * rows_w, rows_w)],
                        idx_v.at[pl.ds(rows_w, rows_w)])
        pltpu.sync_copy(nw_hbm.at[pl.ds(wid * nrows_w, nrows_w)],
                        idx_v.at[pl.ds(2 * rows_w, nrows_w)])

        def chunk_body(s, carry):
            copies = [
                pltpu.async_copy(
                    ctab_hbm.at[idx_v.at[pl.ds(s * _CHUNK, _CHUNK)]],
                    c_v, sem),
                pltpu.async_copy(
                    xtab_hbm.at[idx_v.at[pl.ds(rows_w + s * _CHUNK,
                                               _CHUNK)]],
                    x_v, sem),
            ]
            for q in range(nneg_g):
                copies.append(pltpu.async_copy(
                    xtab_hbm.at[idx_v.at[pl.ds(
                        2 * rows_w + s * _CHUNK * K + q * _IDXW, _IDXW)]],
                    n_v.at[pl.ds(q * _IDXW, _IDXW)], sem))
            for c in copies:
                c.wait()

            def elem_body(e, carry2):
                cvec = [c_v[e, pl.ds(16 * t, 16)] for t in range(4)]
                for j in range(K + 1):
                    if j == 0:
                        r = [x_v[e, pl.ds(16 * t, 16)] for t in range(4)]
                    else:
                        row = e * K + (j - 1)
                        r = [n_v[row, pl.ds(16 * t, 16)] for t in range(4)]
                    p = (cvec[0] * r[0] + cvec[1] * r[1]
                         + cvec[2] * r[2] + cvec[3] * r[3])
                    plsc.store_scatter(tbuf, [iota * _TSTRIDE + j], p)
                s0 = plsc.load_gather(tbuf, [iota])
                s1 = plsc.load_gather(tbuf, [iota + 16])
                for l in range(1, 16):
                    s0 = s0 + plsc.load_gather(tbuf,
                                               [iota + l * _TSTRIDE])
                    s1 = s1 + plsc.load_gather(tbuf,
                                               [iota + l * _TSTRIDE + 16])
                s0 = jnp.where(iota == 0, s0, -s0)
                s1 = -s1
                base = (s * _CHUNK + e) * (K + 1)
                plsc.store_scatter(scores_v, [base + iota], s0)
                plsc.store_scatter(scores_v, [base + 16 + iota], s1)
                return carry2

            lax.fori_loop(0, _CHUNK, elem_body, 0)
            return carry

        lax.fori_loop(0, nchunks, chunk_body, 0)
        pltpu.sync_copy(scores_v.at[pl.ds(0, nscore_w)],
                        out_hbm.at[pl.ds(wid * nscore_w, nscore_w)])

    return k(center_words, context_words, neg_flat, center_table,
             context_table)


def _log_sigmoid(v):
    return jnp.minimum(v, 0.0) - jnp.log1p(jnp.exp(-jnp.abs(v)))


def _tc_loss(scores2d, B):
    def body(s_ref, o_ref):
        o_ref[0, 0] = -jnp.sum(_log_sigmoid(s_ref[...])) / B

    out = pl.pallas_call(
        body,
        out_specs=pl.BlockSpec(memory_space=pltpu.SMEM),
        out_shape=jax.ShapeDtypeStruct((1, 1), jnp.float32),
    )(scores2d)
    return out[0, 0]


def kernel(center_words, context_words, negative_words, center_table,
           context_table):
    B = center_words.shape[0]
    K = negative_words.shape[1]
    D = center_table.shape[1]
    neg_flat = negative_words.reshape(B * K)
    scores = _sc_scores(center_words, context_words, neg_flat,
                        center_table, context_table, K, D)
    scores2d = scores.reshape(B * (K + 1) // 128, 128)
    return _tc_loss(scores2d, B)


# speedup vs baseline: 5.1775x; 1.0249x over previous
import functools

import jax
import jax.numpy as jnp
from jax import lax
from jax.experimental import pallas as pl
from jax.experimental.pallas import tpu as pltpu
from jax.experimental.pallas import tpu_sc as plsc

_NC = 2
_NS = 16
_NW = _NC * _NS
_IDXW = 128
_CHUNK = 32
_TSTRIDE = 25


def _sc_scores(center_words, context_words, neg_flat, center_table,
               context_table, K, D):
    B = center_words.shape[0]
    BK = neg_flat.shape[0]
    rows_w = B // _NW
    nrows_w = BK // _NW
    nchunks = rows_w // _CHUNK
    npairs = nchunks // 2
    nneg_g = _CHUNK * K // _IDXW
    nidx_w = 2 * rows_w + nrows_w
    nscore_w = rows_w * (K + 1)

    mesh = plsc.VectorSubcoreMesh(core_axis_name="c", subcore_axis_name="s")

    @functools.partial(
        pl.kernel,
        out_type=jax.ShapeDtypeStruct((B * (K + 1),), jnp.float32),
        mesh=mesh,
        compiler_params=pltpu.CompilerParams(use_tc_tiling_on_sc=False,
                                             needs_layout_passes=False),
        scratch_types=[
            pltpu.VMEM((nidx_w,), jnp.int32),
            pltpu.VMEM((_CHUNK, D), jnp.float32),
            pltpu.VMEM((_CHUNK, D), jnp.float32),
            pltpu.VMEM((_CHUNK * K, D), jnp.float32),
            pltpu.VMEM((_CHUNK, D), jnp.float32),
            pltpu.VMEM((_CHUNK, D), jnp.float32),
            pltpu.VMEM((_CHUNK * K, D), jnp.float32),
            pltpu.VMEM((16 * _TSTRIDE + 16,), jnp.float32),
            pltpu.VMEM((nscore_w + 16,), jnp.float32),
            pltpu.SemaphoreType.DMA,
            pltpu.SemaphoreType.DMA,
        ],
    )
    def k(cw_hbm, xw_hbm, nw_hbm, ctab_hbm, xtab_hbm, out_hbm,
          idx_v, c_a, x_a, n_a, c_b, x_b, n_b, tbuf, scores_v,
          sem_a, sem_b):
        wid = lax.axis_index("s") * _NC + lax.axis_index("c")
        iota = lax.iota(jnp.int32, 16)

        pltpu.sync_copy(cw_hbm.at[pl.ds(wid * rows_w, rows_w)],
                        idx_v.at[pl.ds(0, rows_w)])
        pltpu.sync_copy(xw_hbm.at[pl.ds(wid * rows_w, rows_w)],
                        idx_v.at[pl.ds(rows_w, rows_w)])
        pltpu.sync_copy(nw_hbm.at[pl.ds(wid * nrows_w, nrows_w)],
                        idx_v.at[pl.ds(2 * rows_w, nrows_w)])

        def descriptors(s, c_v, x_v, n_v, sem):
            ds_list = [
                (ctab_hbm.at[idx_v.at[pl.ds(s * _CHUNK, _CHUNK)]], c_v),
                (xtab_hbm.at[idx_v.at[pl.ds(rows_w + s * _CHUNK, _CHUNK)]],
                 x_v),
            ]
            for q in range(nneg_g):
                ds_list.append((
                    xtab_hbm.at[idx_v.at[pl.ds(
                        2 * rows_w + s * _CHUNK * K + q * _IDXW, _IDXW)]],
                    n_v.at[pl.ds(q * _IDXW, _IDXW)]))
            return [pltpu.make_async_copy(src, dst, sem)
                    for src, dst in ds_list]

        def fire(s, c_v, x_v, n_v, sem):
            for d in descriptors(s, c_v, x_v, n_v, sem):
                d.start()

        def drain(s, c_v, x_v, n_v, sem):
            for d in descriptors(s, c_v, x_v, n_v, sem):
                d.wait()

        def compute(s, c_v, x_v, n_v):
            def elem_body(e, carry2):
                cvec = [c_v[e, pl.ds(16 * t, 16)] for t in range(4)]
                for j in range(K + 1):
                    if j == 0:
                        r = [x_v[e, pl.ds(16 * t, 16)] for t in range(4)]
                    else:
                        row = e * K + (j - 1)
                        r = [n_v[row, pl.ds(16 * t, 16)] for t in range(4)]
                    p = (cvec[0] * r[0] + cvec[1] * r[1]
                         + cvec[2] * r[2] + cvec[3] * r[3])
                    plsc.store_scatter(tbuf, [iota * _TSTRIDE + j], p)
                s0 = plsc.load_gather(tbuf, [iota])
                s1 = plsc.load_gather(tbuf, [iota + 16])
                for l in range(1, 16):
                    s0 = s0 + plsc.load_gather(tbuf,
                                               [iota + l * _TSTRIDE])
                    s1 = s1 + plsc.load_gather(tbuf,
                                               [iota + l * _TSTRIDE + 16])
                s0 = jnp.where(iota == 0, s0, -s0)
                s1 = -s1
                base = (s * _CHUNK + e) * (K + 1)
                plsc.store_scatter(scores_v, [base + iota], s0)
                plsc.store_scatter(scores_v, [base + 16 + iota], s1)
                return carry2

            lax.fori_loop(0, _CHUNK, elem_body, 0)

        fire(0, c_a, x_a, n_a, sem_a)

        def pair_body(u, carry):
            fire(2 * u + 1, c_b, x_b, n_b, sem_b)
            drain(2 * u, c_a, x_a, n_a, sem_a)
            compute(2 * u, c_a, x_a, n_a)

            @pl.when(u < npairs - 1)
            def _():
                fire(2 * u + 2, c_a, x_a, n_a, sem_a)

            drain(2 * u + 1, c_b, x_b, n_b, sem_b)
            compute(2 * u + 1, c_b, x_b, n_b)
            return carry

        lax.fori_loop(0, npairs, pair_body, 0)
        pltpu.sync_copy(scores_v.at[pl.ds(0, nscore_w)],
                        out_hbm.at[pl.ds(wid * nscore_w, nscore_w)])

    return k(center_words, context_words, neg_flat, center_table,
             context_table)


def _log_sigmoid(v):
    return jnp.minimum(v, 0.0) - jnp.log1p(jnp.exp(-jnp.abs(v)))


def _tc_loss(scores2d, B):
    def body(s_ref, o_ref):
        o_ref[0, 0] = -jnp.sum(_log_sigmoid(s_ref[...])) / B

    out = pl.pallas_call(
        body,
        out_specs=pl.BlockSpec(memory_space=pltpu.SMEM),
        out_shape=jax.ShapeDtypeStruct((1, 1), jnp.float32),
    )(scores2d)
    return out[0, 0]


def kernel(center_words, context_words, negative_words, center_table,
           context_table):
    B = center_words.shape[0]
    K = negative_words.shape[1]
    D = center_table.shape[1]
    neg_flat = negative_words.reshape(B * K)
    scores = _sc_scores(center_words, context_words, neg_flat,
                        center_table, context_table, K, D)
    scores2d = scores.reshape(B * (K + 1) // 128, 128)
    return _tc_loss(scores2d, B)
